# Initial kernel scaffold; baseline (speedup 1.0000x reference)
import jax
import jax.numpy as jnp
from jax.experimental import pallas as pl


def kernel(x, img, ws, aff0_w, aff0_b, conv0_w, conv0_b, noise_const0, noise_str0, aff1_w, aff1_b, conv1_w, conv1_b, noise_const1, noise_str1, affrgb_w, affrgb_b, rgb_w, rgb_b):
    raise NotImplementedError("write your pallas kernel here")


# speedup vs baseline: 1.3405x; 1.3405x over previous
import functools
import math

import jax
import jax.numpy as jnp
from jax import lax
from jax.experimental import pallas as pl
from jax.experimental.pallas import tpu as pltpu

_F32 = jnp.float32
_BF16 = jnp.bfloat16


def _style_kernel(wsf_ref, acat_ref, bcat_ref, w0sq_ref, w1sq_ref,
                  s0_ref, d0_ref, s1_ref, d1_ref, s2_ref, *, cin, cout):
    s = jnp.dot(wsf_ref[...], acat_ref[...],
                preferred_element_type=_F32) + bcat_ref[...]
    s0 = s[:, :cin]
    s1 = s[:, cin:cin + cout]
    s0_ref[...] = s0
    s1_ref[...] = s1
    s2_ref[...] = s[:, cin + cout:]
    d0_ref[...] = lax.rsqrt(
        jnp.dot(s0 * s0, w0sq_ref[...], preferred_element_type=_F32) + 1e-8)
    d1_ref[...] = lax.rsqrt(
        jnp.dot(s1 * s1, w1sq_ref[...], preferred_element_type=_F32) + 1e-8)


def _block_kernel(x_ref, img_ref, s0_ref, d0_ref, s1_ref, d1_ref, s2_ref,
                  w0_ref, w1_ref, wrgb_ref, brgb_ref, b0_ref, b1_ref,
                  n0_ref, n1_ref, mxy_ref, msel_ref,
                  xo_ref, io_ref, xp0_ref, xp1_ref, *, H, W, act_gain, clamp):
    HW = H * W
    PAD = W + 1
    cin = x_ref.shape[1]
    cout = xp1_ref.shape[1]

    def col_phases(v):
        sm1 = jnp.concatenate([v[:, :1], v[:, :-1]], axis=1)
        sp1 = jnp.concatenate([v[:, 1:], v[:, -1:]], axis=1)
        cm1 = jnp.where(msel_ref[0:1] > 0.5, v, sm1)
        cp1 = jnp.where(msel_ref[1:2] > 0.5, v, sp1)
        return 0.5 * (cm1 + v), 0.125 * (cm1 + cp1) + 0.75 * v

    def row_phases(v):
        rm1 = jnp.concatenate([v[:, :W], v[:, :-W]], axis=1)
        rp1 = jnp.concatenate([v[:, W:], v[:, -W:]], axis=1)
        return 0.5 * (rm1 + v), 0.125 * (rm1 + rp1) + 0.75 * v

    def up_phases(v):
        ce, co = col_phases(v)
        out = [None] * 4
        for q, plane in ((0, ce), (1, co)):
            pe, po = row_phases(plane)
            out[q] = pe
            out[2 + q] = po
        return out

    xs = (x_ref[0] * s0_ref[0]).astype(_BF16)
    zpi = jnp.zeros((cin, PAD), _BF16)
    xp0_ref[:, :PAD] = zpi
    xp0_ref[:, PAD + HW:] = zpi
    xp0_ref[:, PAD:PAD + HW] = xs
    acc = None
    t = 0
    for dy in (-1, 0, 1):
        for dx in (-1, 0, 1):
            off = PAD + dy * W + dx
            xt = xp0_ref[:, off:off + HW]
            if dx == -1:
                xt = xt * mxy_ref[0:1]
            elif dx == 1:
                xt = xt * mxy_ref[1:2]
            part = jnp.dot(w0_ref[t], xt, preferred_element_type=_F32)
            acc = part if acc is None else acc + part
            t += 1
    h = acc * d0_ref[0]

    zpo = jnp.zeros((cout, PAD), _BF16)
    for ph, plane in enumerate(up_phases(h)):
        r = plane + n0_ref[ph:ph + 1] + b0_ref[...]
        r = jnp.where(r >= 0.0, r, 0.2 * r) * act_gain
        r = jnp.clip(r, -clamp, clamp)
        xp1_ref[ph, :, :PAD] = zpo
        xp1_ref[ph, :, PAD + HW:] = zpo
        xp1_ref[ph, :, PAD:PAD + HW] = (r * s1_ref[0]).astype(_BF16)

    img_up = up_phases(img_ref[0])
    wmod = wrgb_ref[...] * s2_ref[0]

    for p in (0, 1):
        for q in (0, 1):
            ph = 2 * p + q
            acc = None
            for dy in (-1, 0, 1):
                pp = (p + dy) % 2
                di = (p + dy - pp) // 2
                for dx in (-1, 0, 1):
                    qq = (q + dx) % 2
                    dj = (q + dx - qq) // 2
                    t = (dy + 1) * 3 + (dx + 1)
                    off = PAD + di * W + dj
                    xt = xp1_ref[2 * pp + qq, :, off:off + HW]
                    if dj == -1:
                        xt = xt * mxy_ref[0:1]
                    elif dj == 1:
                        xt = xt * mxy_ref[1:2]
                    part = jnp.dot(w1_ref[t], xt, preferred_element_type=_F32)
                    acc = part if acc is None else acc + part
            v = acc * d1_ref[0] + n1_ref[ph:ph + 1] + b1_ref[...]
            v = jnp.where(v >= 0.0, v, 0.2 * v) * act_gain
            v = jnp.clip(v, -clamp, clamp)
            xo_ref[0, ph] = v
            y = jnp.dot(wmod, v, preferred_element_type=_F32)
            io_ref[0, ph] = img_up[ph] + jnp.clip(y + brgb_ref[...],
                                                  -256.0, 256.0)


def kernel(x, img, ws, aff0_w, aff0_b, conv0_w, conv0_b, noise_const0,
           noise_str0, aff1_w, aff1_b, conv1_w, conv1_b, noise_const1,
           noise_str1, affrgb_w, affrgb_b, rgb_w, rgb_b):
    B, Cin, H, W = x.shape
    Cout = conv0_w.shape[0]
    img_ch = rgb_w.shape[0]
    w_dim = aff0_w.shape[1]
    res = 2 * H
    HW = H * W
    act_gain = math.sqrt(2.0)
    clamp = 256.0

    aff_gain = 1.0 / math.sqrt(w_dim)
    rgb_gain = 1.0 / math.sqrt(Cout)
    acat = jnp.zeros((3 * w_dim, Cin + 2 * Cout), _F32)
    acat = acat.at[:w_dim, :Cin].set((aff0_w * aff_gain).T.astype(_F32))
    acat = acat.at[w_dim:2 * w_dim, Cin:Cin + Cout].set(
        (aff1_w * aff_gain).T.astype(_F32))
    acat = acat.at[2 * w_dim:, Cin + Cout:].set(
        (affrgb_w * (aff_gain * rgb_gain)).T.astype(_F32))
    bcat = jnp.concatenate([aff0_b, aff1_b, affrgb_b * rgb_gain]
                           ).reshape(1, Cin + 2 * Cout).astype(_F32)
    w0sq = jnp.sum(conv0_w.astype(_F32) ** 2, axis=(2, 3)).T
    w1sq = jnp.sum(conv1_w.astype(_F32) ** 2, axis=(2, 3)).T
    w0t = jnp.transpose(conv0_w.astype(_F32), (2, 3, 0, 1)
                        ).reshape(9, Cout, Cin).astype(_BF16)
    w1t = jnp.transpose(conv1_w.astype(_F32), (2, 3, 0, 1)
                        ).reshape(9, Cout, Cout).astype(_BF16)
    wrgb = rgb_w.reshape(img_ch, Cout).astype(_F32)
    brgb = rgb_b.reshape(img_ch, 1).astype(_F32)
    b0 = conv0_b.reshape(Cout, 1).astype(_F32)
    b1 = conv1_b.reshape(Cout, 1).astype(_F32)
    n0 = (noise_const0 * noise_str0).astype(_F32).reshape(res, res)
    n1 = (noise_const1 * noise_str1).astype(_F32).reshape(res, res)
    n0p = jnp.stack([n0[p::2, q::2].reshape(HW) for p in (0, 1)
                     for q in (0, 1)])
    n1p = jnp.stack([n1[p::2, q::2].reshape(HW) for p in (0, 1)
                     for q in (0, 1)])
    ww = jnp.arange(HW, dtype=jnp.int32) % W
    mxy = jnp.stack([(ww >= 1).astype(_F32), (ww <= W - 2).astype(_F32)]
                    ).astype(_BF16)
    msel = jnp.stack([(ww == 0).astype(_F32), (ww == W - 1).astype(_F32)])

    s0, d0, s1, d1, s2 = pl.pallas_call(
        functools.partial(_style_kernel, cin=Cin, cout=Cout),
        out_shape=(jax.ShapeDtypeStruct((B, Cin), _F32),
                   jax.ShapeDtypeStruct((B, Cout), _F32),
                   jax.ShapeDtypeStruct((B, Cout), _F32),
                   jax.ShapeDtypeStruct((B, Cout), _F32),
                   jax.ShapeDtypeStruct((B, Cout), _F32)),
    )(ws.reshape(B, 3 * w_dim).astype(_F32), acat, bcat, w0sq, w1sq)

    xo, io = pl.pallas_call(
        functools.partial(_block_kernel, H=H, W=W, act_gain=act_gain,
                          clamp=clamp),
        out_shape=(jax.ShapeDtypeStruct((B, 4, Cout, HW), _F32),
                   jax.ShapeDtypeStruct((B, 4, img_ch, HW), _F32)),
        grid=(B,),
        in_specs=[
            pl.BlockSpec((1, Cin, HW), lambda b: (b, 0, 0)),
            pl.BlockSpec((1, img_ch, HW), lambda b: (b, 0, 0)),
            pl.BlockSpec((1, Cin, 1), lambda b: (b, 0, 0)),
            pl.BlockSpec((1, Cout, 1), lambda b: (b, 0, 0)),
            pl.BlockSpec((1, Cout, 1), lambda b: (b, 0, 0)),
            pl.BlockSpec((1, Cout, 1), lambda b: (b, 0, 0)),
            pl.BlockSpec((1, 1, Cout), lambda b: (b, 0, 0)),
            pl.BlockSpec((9, Cout, Cin), lambda b: (0, 0, 0)),
            pl.BlockSpec((9, Cout, Cout), lambda b: (0, 0, 0)),
            pl.BlockSpec((img_ch, Cout), lambda b: (0, 0)),
            pl.BlockSpec((img_ch, 1), lambda b: (0, 0)),
            pl.BlockSpec((Cout, 1), lambda b: (0, 0)),
            pl.BlockSpec((Cout, 1), lambda b: (0, 0)),
            pl.BlockSpec((4, HW), lambda b: (0, 0)),
            pl.BlockSpec((4, HW), lambda b: (0, 0)),
            pl.BlockSpec((2, HW), lambda b: (0, 0)),
            pl.BlockSpec((2, HW), lambda b: (0, 0)),
        ],
        out_specs=(pl.BlockSpec((1, 4, Cout, HW), lambda b: (b, 0, 0, 0)),
                   pl.BlockSpec((1, 4, img_ch, HW), lambda b: (b, 0, 0, 0))),
        scratch_shapes=[pltpu.VMEM((Cin, HW + 2 * (W + 1)), _BF16),
                        pltpu.VMEM((4, Cout, HW + 2 * (W + 1)), _BF16)],
        compiler_params=pltpu.CompilerParams(
            dimension_semantics=("parallel",)),
    )(x.reshape(B, Cin, HW).astype(_F32), img.reshape(B, img_ch, HW).astype(_F32),
      s0.reshape(B, Cin, 1), d0.reshape(B, Cout, 1), s1.reshape(B, Cout, 1),
      d1.reshape(B, Cout, 1), s2.reshape(B, 1, Cout),
      w0t, w1t, wrgb, brgb, b0, b1, n0p, n1p, mxy, msel)

    x_out = (xo.reshape(B, 2, 2, Cout, H, W)
             .transpose(0, 3, 4, 1, 5, 2).reshape(B, Cout, res, res))
    img_out = (io.reshape(B, 2, 2, img_ch, H, W)
               .transpose(0, 3, 4, 1, 5, 2).reshape(B, img_ch, res, res))
    return x_out, img_out


# speedup vs baseline: 1.4295x; 1.0664x over previous
import functools
import math

import jax
import jax.numpy as jnp
from jax import lax
from jax.experimental import pallas as pl
from jax.experimental.pallas import tpu as pltpu

_F32 = jnp.float32
_BF16 = jnp.bfloat16


def _style_kernel(wsf_ref, acat_ref, bcat_ref, w0sq_ref, w1sq_ref,
                  s0_ref, d0_ref, s1_ref, d1_ref, s2_ref, *, cin, cout):
    s = jnp.dot(wsf_ref[...], acat_ref[...],
                preferred_element_type=_F32) + bcat_ref[...]
    s0 = s[:, :cin]
    s1 = s[:, cin:cin + cout]
    s0_ref[...] = s0
    s1_ref[...] = s1
    s2_ref[...] = s[:, cin + cout:]
    d0_ref[...] = lax.rsqrt(
        jnp.dot(s0 * s0, w0sq_ref[...], preferred_element_type=_F32) + 1e-8)
    d1_ref[...] = lax.rsqrt(
        jnp.dot(s1 * s1, w1sq_ref[...], preferred_element_type=_F32) + 1e-8)


def _block_kernel(x_ref, img_ref, s0_ref, d0_ref, s1_ref, d1_ref, s2_ref,
                  w0_ref, w1_ref, wrgb_ref, brgb_ref, b0_ref, b1_ref,
                  n0_ref, n1_ref, mxy_ref, msel_ref,
                  xo_ref, io_ref, xp0_ref, xp1_ref, *, H, W, act_gain, clamp):
    HW = H * W
    PAD = W + 1
    cin = x_ref.shape[1]
    cout = xp1_ref.shape[1]

    def col_phases(v):
        sm1 = jnp.concatenate([v[:, :1], v[:, :-1]], axis=1)
        sp1 = jnp.concatenate([v[:, 1:], v[:, -1:]], axis=1)
        cm1 = jnp.where(msel_ref[0:1] > 0.5, v, sm1)
        cp1 = jnp.where(msel_ref[1:2] > 0.5, v, sp1)
        return 0.5 * (cm1 + v), 0.125 * (cm1 + cp1) + 0.75 * v

    def row_phases(v):
        rm1 = jnp.concatenate([v[:, :W], v[:, :-W]], axis=1)
        rp1 = jnp.concatenate([v[:, W:], v[:, -W:]], axis=1)
        return 0.5 * (rm1 + v), 0.125 * (rm1 + rp1) + 0.75 * v

    def up_phases(v):
        ce, co = col_phases(v)
        out = [None] * 4
        for q, plane in ((0, ce), (1, co)):
            pe, po = row_phases(plane)
            out[q] = pe
            out[2 + q] = po
        return out

    wmod0 = d0_ref[0] * s0_ref[0]
    xs = x_ref[0].astype(_BF16)
    zpi = jnp.zeros((cin, PAD), _BF16)
    xp0_ref[:, :PAD] = zpi
    xp0_ref[:, PAD + HW:] = zpi
    xp0_ref[:, PAD:PAD + HW] = xs
    acc = None
    t = 0
    for dy in (-1, 0, 1):
        for dx in (-1, 0, 1):
            off = PAD + dy * W + dx
            xt = xp0_ref[:, off:off + HW]
            if dx == -1:
                xt = xt * mxy_ref[0:1]
            elif dx == 1:
                xt = xt * mxy_ref[1:2]
            we = (w0_ref[t] * wmod0).astype(_BF16)
            part = jnp.dot(we, xt, preferred_element_type=_F32)
            acc = part if acc is None else acc + part
            t += 1

    zpo = jnp.zeros((cout, PAD), _BF16)
    for ph, plane in enumerate(up_phases(acc.astype(_BF16))):
        r = plane + n0_ref[ph:ph + 1] + b0_ref[...]
        r = jnp.where(r >= 0.0, r, jnp.bfloat16(0.2) * r) * jnp.bfloat16(act_gain)
        r = jnp.clip(r, jnp.bfloat16(-clamp), jnp.bfloat16(clamp))
        xp1_ref[ph, :, :PAD] = zpo
        xp1_ref[ph, :, PAD + HW:] = zpo
        xp1_ref[ph, :, PAD:PAD + HW] = r

    img_up = up_phases(img_ref[0].astype(_BF16))
    wmod1 = d1_ref[0] * s1_ref[0]
    wmod_rgb = wrgb_ref[...] * s2_ref[0]

    w1e = [(w1_ref[t] * wmod1).astype(_BF16) for t in range(9)]
    for p in (0, 1):
        for q in (0, 1):
            ph = 2 * p + q
            acc = None
            for dy in (-1, 0, 1):
                pp = (p + dy) % 2
                di = (p + dy - pp) // 2
                for dx in (-1, 0, 1):
                    qq = (q + dx) % 2
                    dj = (q + dx - qq) // 2
                    t = (dy + 1) * 3 + (dx + 1)
                    off = PAD + di * W + dj
                    xt = xp1_ref[2 * pp + qq, :, off:off + HW]
                    if dj == -1:
                        xt = xt * mxy_ref[0:1]
                    elif dj == 1:
                        xt = xt * mxy_ref[1:2]
                    part = jnp.dot(w1e[t], xt, preferred_element_type=_F32)
                    acc = part if acc is None else acc + part
            v = acc + n1_ref[ph:ph + 1] + b1_ref[...]
            v = jnp.where(v >= 0.0, v, 0.2 * v) * act_gain
            v = jnp.clip(v, -clamp, clamp)
            xo_ref[0, ph] = v
            y = jnp.dot(wmod_rgb, v, preferred_element_type=_F32)
            io_ref[0, ph] = (img_up[ph].astype(_F32)
                             + jnp.clip(y + brgb_ref[...], -256.0, 256.0))


def kernel(x, img, ws, aff0_w, aff0_b, conv0_w, conv0_b, noise_const0,
           noise_str0, aff1_w, aff1_b, conv1_w, conv1_b, noise_const1,
           noise_str1, affrgb_w, affrgb_b, rgb_w, rgb_b):
    B, Cin, H, W = x.shape
    Cout = conv0_w.shape[0]
    img_ch = rgb_w.shape[0]
    w_dim = aff0_w.shape[1]
    res = 2 * H
    HW = H * W
    act_gain = math.sqrt(2.0)
    clamp = 256.0

    aff_gain = 1.0 / math.sqrt(w_dim)
    rgb_gain = 1.0 / math.sqrt(Cout)
    acat = jnp.zeros((3 * w_dim, Cin + 2 * Cout), _F32)
    acat = acat.at[:w_dim, :Cin].set((aff0_w * aff_gain).T.astype(_F32))
    acat = acat.at[w_dim:2 * w_dim, Cin:Cin + Cout].set(
        (aff1_w * aff_gain).T.astype(_F32))
    acat = acat.at[2 * w_dim:, Cin + Cout:].set(
        (affrgb_w * (aff_gain * rgb_gain)).T.astype(_F32))
    bcat = jnp.concatenate([aff0_b, aff1_b, affrgb_b * rgb_gain]
                           ).reshape(1, Cin + 2 * Cout).astype(_F32)
    w0sq = jnp.sum(conv0_w.astype(_F32) ** 2, axis=(2, 3)).T
    w1sq = jnp.sum(conv1_w.astype(_F32) ** 2, axis=(2, 3)).T
    w0t = jnp.transpose(conv0_w.astype(_F32), (2, 3, 0, 1)).reshape(9, Cout, Cin)
    w1t = jnp.transpose(conv1_w.astype(_F32), (2, 3, 0, 1)).reshape(9, Cout, Cout)
    wrgb = rgb_w.reshape(img_ch, Cout).astype(_F32)
    brgb = rgb_b.reshape(img_ch, 1).astype(_F32)
    b0 = conv0_b.reshape(Cout, 1).astype(_BF16)
    b1 = conv1_b.reshape(Cout, 1).astype(_F32)
    n0 = (noise_const0 * noise_str0).astype(_F32).reshape(res, res)
    n1 = (noise_const1 * noise_str1).astype(_F32).reshape(res, res)
    n0p = jnp.stack([n0[p::2, q::2].reshape(HW) for p in (0, 1)
                     for q in (0, 1)]).astype(_BF16)
    n1p = jnp.stack([n1[p::2, q::2].reshape(HW) for p in (0, 1)
                     for q in (0, 1)])
    ww = jnp.arange(HW, dtype=jnp.int32) % W
    mxy = jnp.stack([(ww >= 1).astype(_F32), (ww <= W - 2).astype(_F32)]
                    ).astype(_BF16)
    msel = jnp.stack([(ww == 0).astype(_F32), (ww == W - 1).astype(_F32)]
                     ).astype(_BF16)

    s0, d0, s1, d1, s2 = pl.pallas_call(
        functools.partial(_style_kernel, cin=Cin, cout=Cout),
        out_shape=(jax.ShapeDtypeStruct((B, Cin), _F32),
                   jax.ShapeDtypeStruct((B, Cout), _F32),
                   jax.ShapeDtypeStruct((B, Cout), _F32),
                   jax.ShapeDtypeStruct((B, Cout), _F32),
                   jax.ShapeDtypeStruct((B, Cout), _F32)),
    )(ws.reshape(B, 3 * w_dim).astype(_F32), acat, bcat, w0sq, w1sq)

    xo, io = pl.pallas_call(
        functools.partial(_block_kernel, H=H, W=W, act_gain=act_gain,
                          clamp=clamp),
        out_shape=(jax.ShapeDtypeStruct((B, 4, Cout, HW), _F32),
                   jax.ShapeDtypeStruct((B, 4, img_ch, HW), _F32)),
        grid=(B,),
        in_specs=[
            pl.BlockSpec((1, Cin, HW), lambda b: (b, 0, 0)),
            pl.BlockSpec((1, img_ch, HW), lambda b: (b, 0, 0)),
            pl.BlockSpec((1, 1, Cin), lambda b: (b, 0, 0)),
            pl.BlockSpec((1, Cout, 1), lambda b: (b, 0, 0)),
            pl.BlockSpec((1, 1, Cout), lambda b: (b, 0, 0)),
            pl.BlockSpec((1, Cout, 1), lambda b: (b, 0, 0)),
            pl.BlockSpec((1, 1, Cout), lambda b: (b, 0, 0)),
            pl.BlockSpec((9, Cout, Cin), lambda b: (0, 0, 0)),
            pl.BlockSpec((9, Cout, Cout), lambda b: (0, 0, 0)),
            pl.BlockSpec((img_ch, Cout), lambda b: (0, 0)),
            pl.BlockSpec((img_ch, 1), lambda b: (0, 0)),
            pl.BlockSpec((Cout, 1), lambda b: (0, 0)),
            pl.BlockSpec((Cout, 1), lambda b: (0, 0)),
            pl.BlockSpec((4, HW), lambda b: (0, 0)),
            pl.BlockSpec((4, HW), lambda b: (0, 0)),
            pl.BlockSpec((2, HW), lambda b: (0, 0)),
            pl.BlockSpec((2, HW), lambda b: (0, 0)),
        ],
        out_specs=(pl.BlockSpec((1, 4, Cout, HW), lambda b: (b, 0, 0, 0)),
                   pl.BlockSpec((1, 4, img_ch, HW), lambda b: (b, 0, 0, 0))),
        scratch_shapes=[pltpu.VMEM((Cin, HW + 2 * (W + 1)), _BF16),
                        pltpu.VMEM((4, Cout, HW + 2 * (W + 1)), _BF16)],
        compiler_params=pltpu.CompilerParams(
            dimension_semantics=("parallel",)),
    )(x.reshape(B, Cin, HW).astype(_F32), img.reshape(B, img_ch, HW).astype(_F32),
      s0.reshape(B, 1, Cin), d0.reshape(B, Cout, 1), s1.reshape(B, 1, Cout),
      d1.reshape(B, Cout, 1), s2.reshape(B, 1, Cout),
      w0t, w1t, wrgb, brgb, b0, b1, n0p, n1p, mxy, msel)

    x_out = (xo.reshape(B, 2, 2, Cout, H, W)
             .transpose(0, 3, 4, 1, 5, 2).reshape(B, Cout, res, res))
    img_out = (io.reshape(B, 2, 2, img_ch, H, W)
               .transpose(0, 3, 4, 1, 5, 2).reshape(B, img_ch, res, res))
    return x_out, img_out
